# Initial kernel scaffold; baseline (speedup 1.0000x reference)
import jax
import jax.numpy as jnp
from jax.experimental import pallas as pl


def kernel(mel2ph, dur):
    raise NotImplementedError("write your pallas kernel here")


# speedup vs baseline: 26.7506x; 26.7506x over previous
import functools

import jax
import jax.numpy as jnp
from jax import lax
from jax.experimental import pallas as pl
from jax.experimental.pallas import tpu as pltpu
from jax.experimental.pallas import tpu_sc as plsc

B = 16
T_SPEECH = 4096
T_TXT = 512
L = 16
CHUNKS = T_SPEECH // L
ROWS_PER_CORE = 8


def _body(m_hbm, d_hbm, out_hbm, m_v, d_v, o_v):
    c = lax.axis_index("c")
    s = lax.axis_index("s")
    row = c * ROWS_PER_CORE + s

    @pl.when(s < ROWS_PER_CORE)
    def _():
        pltpu.sync_copy(m_hbm.at[row], m_v.at[pl.ds(0, T_SPEECH)])
        pltpu.sync_copy(d_hbm.at[row], d_v)
        m_v[pl.ds(T_SPEECH, L)] = jnp.full((L,), T_TXT, jnp.int32)

        def step(k, carry):
            idx = m_v[pl.ds(k * L, L)]
            idxn = m_v[pl.ds(k * L + 1, L)]
            g = plsc.load_gather(d_v, [jnp.maximum(idx - 1, 0)])
            pos = idx > 0
            mel2dur = jnp.where(pos, g, jnp.float32(1.0))
            delta = jnp.where(idxn > idx, jnp.float32(1.0) - mel2dur,
                              jnp.float32(1.0))
            csum = plsc.cumsum(delta)
            excl = csum - delta + carry
            o_v[pl.ds(k * L, L)] = jnp.where(
                pos, excl / mel2dur, jnp.float32(0.0))
            return carry + jnp.sum(delta)

        lax.fori_loop(0, CHUNKS, step, jnp.float32(0.0))
        pltpu.sync_copy(o_v, out_hbm.at[row])


@jax.jit
def _run(mel2ph, dur):
    mesh = plsc.VectorSubcoreMesh(core_axis_name="c", subcore_axis_name="s")
    f = pl.kernel(
        _body,
        out_type=jax.ShapeDtypeStruct((B, T_SPEECH), jnp.float32),
        mesh=mesh,
        compiler_params=pltpu.CompilerParams(needs_layout_passes=False),
        scratch_types=[
            pltpu.VMEM((T_SPEECH + L,), jnp.int32),
            pltpu.VMEM((T_TXT,), jnp.float32),
            pltpu.VMEM((T_SPEECH,), jnp.float32),
        ],
    )
    return f(mel2ph, dur)


def kernel(mel2ph, dur):
    return _run(mel2ph.astype(jnp.int32), dur)


# speedup vs baseline: 30.8896x; 1.1547x over previous
import functools

import jax
import jax.numpy as jnp
from jax import lax
from jax.experimental import pallas as pl
from jax.experimental.pallas import tpu as pltpu
from jax.experimental.pallas import tpu_sc as plsc

B = 16
T_SPEECH = 4096
T_TXT = 512
L = 16
CHUNKS = T_SPEECH // L
ROWS_PER_CORE = 8


def _body(m_hbm, d_hbm, out_hbm, m_v, d_v, o_v):
    c = lax.axis_index("c")
    s = lax.axis_index("s")
    row = c * ROWS_PER_CORE + s

    @pl.when(s < ROWS_PER_CORE)
    def _():
        pltpu.sync_copy(m_hbm.at[row], m_v.at[pl.ds(0, T_SPEECH)])
        pltpu.sync_copy(d_hbm.at[row], d_v)
        m_v[pl.ds(T_SPEECH, L)] = jnp.full((L,), T_TXT, jnp.int32)

        @plsc.parallel_loop(0, CHUNKS, unroll=4, carry=jnp.float32(0.0))
        def _loop(k, carry):
            idx = m_v[pl.ds(k * L, L)]
            idxn = m_v[pl.ds(k * L + 1, L)]
            g = plsc.load_gather(d_v, [jnp.maximum(idx - 1, 0)])
            pos = idx > 0
            mel2dur = jnp.where(pos, g, jnp.float32(1.0))
            delta = jnp.where(idxn > idx, jnp.float32(1.0) - mel2dur,
                              jnp.float32(1.0))
            csum = plsc.cumsum(delta)
            excl = csum - delta + carry
            o_v[pl.ds(k * L, L)] = jnp.where(
                pos, excl / mel2dur, jnp.float32(0.0))
            return carry + jnp.sum(delta)
        pltpu.sync_copy(o_v, out_hbm.at[row])


@jax.jit
def _run(mel2ph, dur):
    mesh = plsc.VectorSubcoreMesh(core_axis_name="c", subcore_axis_name="s")
    f = pl.kernel(
        _body,
        out_type=jax.ShapeDtypeStruct((B, T_SPEECH), jnp.float32),
        mesh=mesh,
        compiler_params=pltpu.CompilerParams(needs_layout_passes=False),
        scratch_types=[
            pltpu.VMEM((T_SPEECH + L,), jnp.int32),
            pltpu.VMEM((T_TXT,), jnp.float32),
            pltpu.VMEM((T_SPEECH,), jnp.float32),
        ],
    )
    return f(mel2ph, dur)


def kernel(mel2ph, dur):
    return _run(mel2ph.astype(jnp.int32), dur)
